# baseline (speedup 1.0000x reference)
import functools
import math

import jax
import jax.numpy as jnp
from jax import lax
from jax.experimental import pallas as pl
from jax.experimental.pallas import tpu as pltpu
from jax.experimental.pallas import tpu_sc as plsc

N, E, D, R = 10000, 320000, 128, 16

NUM_CORES = 2
NUM_SUBCORES = 16
NW = NUM_CORES * NUM_SUBCORES
E_PER_W = E // NW
B = 400
N_CHUNKS = E_PER_W // B
GROUPS = B // 16


def _proj_body(emb_ref, wk_ref, wq_ref, k_ref, q_ref):
    scale = 1.0 / math.sqrt(D)
    dn = (((1,), (1,)), ((), ()))
    k = lax.dot_general(emb_ref[...], wk_ref[...], dn,
                        precision=lax.Precision.HIGHEST,
                        preferred_element_type=jnp.float32)
    k_ref[...] = k * scale
    q_ref[...] = lax.dot_general(emb_ref[...], wq_ref[...], dn,
                                 precision=lax.Precision.HIGHEST,
                                 preferred_element_type=jnp.float32)


def _project(node_emb, tokeys, toqueries):
    return pl.pallas_call(
        _proj_body,
        out_shape=[jax.ShapeDtypeStruct((N, D), jnp.float32),
                   jax.ShapeDtypeStruct((N, D), jnp.float32)],
    )(node_emb, tokeys, toqueries)


def _sc_body(k_hbm, q_hbm, emb_hbm, si_hbm, oi_hbm, p_hbm,
             rel_hbm, dots_hbm, newemb_hbm,
             si_v, oi_v, p_v, krows, qrows, rel_v, dots_v, sem, sem2):
    wid = lax.axis_index("s") * NUM_CORES + lax.axis_index("c")
    pltpu.sync_copy(rel_hbm, rel_v)

    def chunk_body(i, _):
        base = wid * E_PER_W + i * B
        pltpu.sync_copy(si_hbm.at[pl.ds(base, B)], si_v)
        pltpu.sync_copy(oi_hbm.at[pl.ds(base, B)], oi_v)
        pltpu.sync_copy(p_hbm.at[pl.ds(base, B)], p_v)
        cp_k = pltpu.async_copy(k_hbm.at[si_v], krows, sem)
        cp_q = pltpu.async_copy(q_hbm.at[oi_v], qrows, sem2)
        cp_k.wait()
        cp_q.wait()

        def group_body(j, _):
            rows = j * 16 + lax.iota(jnp.int32, 16)
            p_vec = p_v[pl.ds(j * 16, 16)]
            zero = jnp.zeros((16,), jnp.float32)

            @plsc.parallel_loop(0, D, step=16, carry=(zero, zero, zero, zero))
            def d_loop(d, accs):
                accs = list(accs)
                for u in range(16):
                    col = jnp.full((16,), d + u, jnp.int32)
                    kv = plsc.load_gather(krows, [rows, col])
                    qv = plsc.load_gather(qrows, [rows, col])
                    rv = plsc.load_gather(rel_v, [p_vec, col])
                    accs[u % 4] = accs[u % 4] + kv * rv * qv
                return tuple(accs)

            a0, a1, a2, a3 = d_loop
            dots_v[pl.ds(j * 16, 16)] = (a0 + a1) + (a2 + a3)
            return 0

        lax.fori_loop(0, GROUPS, group_body, 0)
        pltpu.sync_copy(dots_v, dots_hbm.at[pl.ds(base, B)])

        pltpu.async_copy(emb_hbm.at[oi_v], krows, sem).wait()
        pltpu.sync_copy(krows, newemb_hbm.at[pl.ds(base, B)])
        return 0

    lax.fori_loop(0, N_CHUNKS, chunk_body, 0)


def _sc_call(k_tab, q_tab, node_emb, si, oi, p, relations):
    mesh = plsc.VectorSubcoreMesh(core_axis_name="c", subcore_axis_name="s",
                                  num_cores=NUM_CORES,
                                  num_subcores=NUM_SUBCORES)
    f = pl.kernel(
        _sc_body,
        out_type=[jax.ShapeDtypeStruct((E,), jnp.float32),
                  jax.ShapeDtypeStruct((E, D), jnp.float32)],
        mesh=mesh,
        compiler_params=pltpu.CompilerParams(needs_layout_passes=False),
        scratch_types=[
            pltpu.VMEM((B,), jnp.int32),
            pltpu.VMEM((B,), jnp.int32),
            pltpu.VMEM((B,), jnp.int32),
            pltpu.VMEM((B, D), jnp.float32),
            pltpu.VMEM((B, D), jnp.float32),
            pltpu.VMEM((R, D), jnp.float32),
            pltpu.VMEM((B,), jnp.float32),
            pltpu.SemaphoreType.DMA,
            pltpu.SemaphoreType.DMA,
        ],
    )
    return f(k_tab, q_tab, node_emb, si, oi, p, relations)


def kernel(node_emb, edge_index, edge_type, relations, tokeys, toqueries):
    k_tab, q_tab = _project(node_emb, tokeys, toqueries)
    si = edge_index[0]
    oi = edge_index[1]
    dots, new_node_emb = _sc_call(k_tab, q_tab, node_emb, si, oi,
                                  edge_type, relations)
    return dots, new_node_emb


# speedup vs baseline: 3.6774x; 3.6774x over previous
import functools
import math

import jax
import jax.numpy as jnp
from jax import lax
from jax.experimental import pallas as pl
from jax.experimental.pallas import tpu as pltpu
from jax.experimental.pallas import tpu_sc as plsc

N, E, D, R = 10000, 320000, 128, 16

NUM_CORES = 2
NUM_SUBCORES = 16
NW = NUM_CORES * NUM_SUBCORES
E_PER_W = E // NW
B = 400
N_CHUNKS = E_PER_W // B


def _proj_body(emb_ref, wk_ref, wq_ref, k_ref, q_ref):
    scale = 1.0 / math.sqrt(D)
    dn = (((1,), (1,)), ((), ()))
    k = lax.dot_general(emb_ref[...], wk_ref[...], dn,
                        precision=lax.Precision.HIGHEST,
                        preferred_element_type=jnp.float32)
    k_ref[...] = k * scale
    q_ref[...] = lax.dot_general(emb_ref[...], wq_ref[...], dn,
                                 precision=lax.Precision.HIGHEST,
                                 preferred_element_type=jnp.float32)


def _project(node_emb, tokeys, toqueries):
    return pl.pallas_call(
        _proj_body,
        out_shape=[jax.ShapeDtypeStruct((N, D), jnp.float32),
                   jax.ShapeDtypeStruct((N, D), jnp.float32)],
    )(node_emb, tokeys, toqueries)


def _expand_body(k_ref, rel_ref, a_ref):
    r = pl.program_id(0)
    a_ref[...] = k_ref[...] * rel_ref[r, :][None, :]


def _expand(k_tab, relations):
    return pl.pallas_call(
        _expand_body,
        grid=(R,),
        in_specs=[pl.BlockSpec((N, D), lambda r: (0, 0)),
                  pl.BlockSpec((R, D), lambda r: (0, 0))],
        out_specs=pl.BlockSpec((N, D), lambda r: (r, 0)),
        out_shape=jax.ShapeDtypeStruct((R * N, D), jnp.float32),
    )(k_tab, relations)


def _sc_body(a_hbm, q_hbm, emb_hbm, si_hbm, oi_hbm, p_hbm,
             dots_hbm, newemb_hbm,
             si_v, oi_v, p_v, ai_v, arows, qrows, tmp_v, dots_v, sem, sem2):
    wid = lax.axis_index("s") * NUM_CORES + lax.axis_index("c")

    def chunk_body(i, _):
        base = wid * E_PER_W + i * B
        pltpu.sync_copy(si_hbm.at[pl.ds(base, B)], si_v)
        pltpu.sync_copy(oi_hbm.at[pl.ds(base, B)], oi_v)
        pltpu.sync_copy(p_hbm.at[pl.ds(base, B)], p_v)

        @plsc.parallel_loop(0, B, step=16)
        def idx_body(t):
            ai_v[pl.ds(t, 16)] = (p_v[pl.ds(t, 16)] * N + si_v[pl.ds(t, 16)])

        cp_a = pltpu.async_copy(a_hbm.at[ai_v], arows, sem)
        cp_q = pltpu.async_copy(q_hbm.at[oi_v], qrows, sem2)
        cp_a.wait()
        cp_q.wait()

        lanes = lax.iota(jnp.int32, 16)

        @plsc.parallel_loop(0, B, step=16)
        def e_body(t):
            for l in range(16):
                e = t + l
                acc = arows[e, pl.ds(0, 16)] * qrows[e, pl.ds(0, 16)]
                for c in range(1, 8):
                    acc = acc + (arows[e, pl.ds(c * 16, 16)] *
                                 qrows[e, pl.ds(c * 16, 16)])
                tmp_v[pl.ds(e * 17, 16)] = acc
            base17 = (t + lanes) * 17
            tot = plsc.load_gather(tmp_v, [base17])
            for c in range(1, 16):
                tot = tot + plsc.load_gather(tmp_v, [base17 + c])
            dots_v[pl.ds(t, 16)] = tot

        pltpu.sync_copy(dots_v, dots_hbm.at[pl.ds(base, B)])

        pltpu.async_copy(emb_hbm.at[oi_v], arows, sem).wait()
        pltpu.sync_copy(arows, newemb_hbm.at[pl.ds(base, B)])
        return 0

    lax.fori_loop(0, N_CHUNKS, chunk_body, 0)


def _sc_call(a_tab, q_tab, node_emb, si, oi, p):
    mesh = plsc.VectorSubcoreMesh(core_axis_name="c", subcore_axis_name="s",
                                  num_cores=NUM_CORES,
                                  num_subcores=NUM_SUBCORES)
    f = pl.kernel(
        _sc_body,
        out_type=[jax.ShapeDtypeStruct((E,), jnp.float32),
                  jax.ShapeDtypeStruct((E, D), jnp.float32)],
        mesh=mesh,
        compiler_params=pltpu.CompilerParams(needs_layout_passes=False),
        scratch_types=[
            pltpu.VMEM((B,), jnp.int32),
            pltpu.VMEM((B,), jnp.int32),
            pltpu.VMEM((B,), jnp.int32),
            pltpu.VMEM((B,), jnp.int32),
            pltpu.VMEM((B, D), jnp.float32),
            pltpu.VMEM((B, D), jnp.float32),
            pltpu.VMEM((B * 17,), jnp.float32),
            pltpu.VMEM((B,), jnp.float32),
            pltpu.SemaphoreType.DMA,
            pltpu.SemaphoreType.DMA,
        ],
    )
    return f(a_tab, q_tab, node_emb, si, oi, p)


def kernel(node_emb, edge_index, edge_type, relations, tokeys, toqueries):
    k_tab, q_tab = _project(node_emb, tokeys, toqueries)
    a_tab = _expand(k_tab, relations)
    si = edge_index[0]
    oi = edge_index[1]
    dots, new_node_emb = _sc_call(a_tab, q_tab, node_emb, si, oi, edge_type)
    return dots, new_node_emb


# speedup vs baseline: 6.2322x; 1.6947x over previous
import functools
import math

import jax
import jax.numpy as jnp
from jax import lax
from jax.experimental import pallas as pl
from jax.experimental.pallas import tpu as pltpu
from jax.experimental.pallas import tpu_sc as plsc

N, E, D, R = 10000, 320000, 128, 16

NUM_CORES = 2
NUM_SUBCORES = 16
NW = NUM_CORES * NUM_SUBCORES
E_PER_W = E // NW
B = 80
N_CHUNKS = E_PER_W // B


def _proj_body(emb_ref, wk_ref, wq_ref, k_ref, q_ref):
    scale = 1.0 / math.sqrt(D)
    dn = (((1,), (1,)), ((), ()))
    k = lax.dot_general(emb_ref[...], wk_ref[...], dn,
                        precision=lax.Precision.HIGHEST,
                        preferred_element_type=jnp.float32)
    k_ref[...] = k * scale
    q_ref[...] = lax.dot_general(emb_ref[...], wq_ref[...], dn,
                                 precision=lax.Precision.HIGHEST,
                                 preferred_element_type=jnp.float32)


def _project(node_emb, tokeys, toqueries):
    return pl.pallas_call(
        _proj_body,
        out_shape=[jax.ShapeDtypeStruct((N, D), jnp.float32),
                   jax.ShapeDtypeStruct((N, D), jnp.float32)],
    )(node_emb, tokeys, toqueries)


def _expand_body(k_ref, rel_ref, a_ref):
    r = pl.program_id(0)
    a_ref[...] = k_ref[...] * rel_ref[r, :][None, :]


def _expand(k_tab, relations):
    return pl.pallas_call(
        _expand_body,
        grid=(R,),
        in_specs=[pl.BlockSpec((N, D), lambda r: (0, 0)),
                  pl.BlockSpec((R, D), lambda r: (0, 0))],
        out_specs=pl.BlockSpec((N, D), lambda r: (r, 0)),
        out_shape=jax.ShapeDtypeStruct((R * N, D), jnp.float32),
    )(k_tab, relations)


def _sc_body(a_hbm, q_hbm, emb_hbm, si_hbm, oi_hbm, p_hbm,
             dots_hbm, newemb_hbm,
             ai_v, oi_v, p_v,
             ar0, ar1, qr0, qr1, er0, er1,
             tmp_v, dots_v, gs0, gs1, ws0, ws1):
    wid = lax.axis_index("s") * NUM_CORES + lax.axis_index("c")
    wbase = wid * E_PER_W
    lanes = lax.iota(jnp.int32, 16)

    pltpu.sync_copy(si_hbm.at[pl.ds(wbase, E_PER_W)], ai_v)
    pltpu.sync_copy(oi_hbm.at[pl.ds(wbase, E_PER_W)], oi_v)
    pltpu.sync_copy(p_hbm.at[pl.ds(wbase, E_PER_W)], p_v)

    @plsc.parallel_loop(0, E_PER_W, step=16)
    def idx_body(t):
        ai_v[pl.ds(t, 16)] = p_v[pl.ds(t, 16)] * N + ai_v[pl.ds(t, 16)]

    slots = ((ar0, qr0, er0, gs0, ws0), (ar1, qr1, er1, gs1, ws1))

    def fire(g, slot):
        ar, qr, er, gs, _ = slot
        off = g * B
        pltpu.async_copy(a_hbm.at[ai_v.at[pl.ds(off, B)]], ar, gs)
        pltpu.async_copy(q_hbm.at[oi_v.at[pl.ds(off, B)]], qr, gs)
        pltpu.async_copy(emb_hbm.at[oi_v.at[pl.ds(off, B)]], er, gs)

    def drain_gathers(slot):
        ar, qr, er, gs, _ = slot
        dummy = a_hbm.at[pl.ds(0, B)]
        pltpu.make_async_copy(dummy, ar, gs).wait()
        pltpu.make_async_copy(dummy, qr, gs).wait()
        pltpu.make_async_copy(dummy, er, gs).wait()

    def fire_wb(g, slot):
        _, _, er, _, ws = slot
        pltpu.async_copy(er, newemb_hbm.at[pl.ds(wbase + g * B, B)], ws)

    def wait_wb(g, slot):
        _, _, er, _, ws = slot
        pltpu.make_async_copy(er, newemb_hbm.at[pl.ds(wbase + g * B, B)],
                              ws).wait()

    def compute(g, slot):
        ar, qr, _, _, _ = slot
        goff = g * B

        @plsc.parallel_loop(0, B, step=16)
        def e_body(t):
            for l in range(16):
                e = t + l
                acc = ar[e, pl.ds(0, 16)] * qr[e, pl.ds(0, 16)]
                for c in range(1, 8):
                    acc = acc + (ar[e, pl.ds(c * 16, 16)] *
                                 qr[e, pl.ds(c * 16, 16)])
                tmp_v[pl.ds(e * 17, 16)] = acc
            base17 = (t + lanes) * 17
            tot = plsc.load_gather(tmp_v, [base17])
            for c in range(1, 16):
                tot = tot + plsc.load_gather(tmp_v, [base17 + c])
            dots_v[pl.ds(goff + t, 16)] = tot

    def step(g, slot):
        drain_gathers(slot)
        fire_wb(g, slot)
        compute(g, slot)
        wait_wb(g, slot)

    fire(0, slots[0])
    fire(1, slots[1])

    def pair_body(v, _):
        g = v * 2
        for par in range(2):
            slot = slots[par]
            drain_gathers(slot)
            fire_wb(g + par, slot)
            compute(g + par, slot)
            wait_wb(g + par, slot)
            fire(g + par + 2, slot)
        return 0

    lax.fori_loop(0, (N_CHUNKS - 3) // 2, pair_body, 0)
    g_tail = N_CHUNKS - 3
    for i, g in enumerate(range(g_tail, N_CHUNKS)):
        slot = slots[g % 2]
        drain_gathers(slot)
        fire_wb(g, slot)
        compute(g, slot)
        wait_wb(g, slot)
        if g + 2 < N_CHUNKS:
            fire(g + 2, slot)

    pltpu.sync_copy(dots_v, dots_hbm.at[pl.ds(wbase, E_PER_W)])


def _sc_call(a_tab, q_tab, node_emb, si, oi, p):
    mesh = plsc.VectorSubcoreMesh(core_axis_name="c", subcore_axis_name="s",
                                  num_cores=NUM_CORES,
                                  num_subcores=NUM_SUBCORES)
    f = pl.kernel(
        _sc_body,
        out_type=[jax.ShapeDtypeStruct((E,), jnp.float32),
                  jax.ShapeDtypeStruct((E, D), jnp.float32)],
        mesh=mesh,
        compiler_params=pltpu.CompilerParams(needs_layout_passes=False),
        scratch_types=[
            pltpu.VMEM((E_PER_W,), jnp.int32),
            pltpu.VMEM((E_PER_W,), jnp.int32),
            pltpu.VMEM((E_PER_W,), jnp.int32),
            pltpu.VMEM((B, D), jnp.float32),
            pltpu.VMEM((B, D), jnp.float32),
            pltpu.VMEM((B, D), jnp.float32),
            pltpu.VMEM((B, D), jnp.float32),
            pltpu.VMEM((B, D), jnp.float32),
            pltpu.VMEM((B, D), jnp.float32),
            pltpu.VMEM((B * 17,), jnp.float32),
            pltpu.VMEM((E_PER_W,), jnp.float32),
            pltpu.SemaphoreType.DMA,
            pltpu.SemaphoreType.DMA,
            pltpu.SemaphoreType.DMA,
            pltpu.SemaphoreType.DMA,
        ],
    )
    return f(a_tab, q_tab, node_emb, si, oi, p)


def kernel(node_emb, edge_index, edge_type, relations, tokeys, toqueries):
    k_tab, q_tab = _project(node_emb, tokeys, toqueries)
    a_tab = _expand(k_tab, relations)
    si = edge_index[0]
    oi = edge_index[1]
    dots, new_node_emb = _sc_call(a_tab, q_tab, node_emb, si, oi, edge_type)
    return dots, new_node_emb
